# baseline (speedup 1.0000x reference)
import functools

import jax
import jax.numpy as jnp
from jax import lax
from jax.experimental import pallas as pl
from jax.experimental.pallas import tpu as pltpu
from jax.experimental.pallas import tpu_sc as plsc

N = 10000
NW = 32
NPW = 320
NPAD = NW * NPW
CH = 1024
K = 64
NC = 2
NS = 16
_DEBUG_STAGE = 0

_mesh = lambda: plsc.VectorSubcoreMesh(
    core_axis_name="c", subcore_axis_name="s", num_cores=NC, num_subcores=NS)
_SC_PARAMS = pltpu.CompilerParams(needs_layout_passes=False)


def _wid():
    return lax.axis_index("s") * NC + lax.axis_index("c")


def _iota16():
    return lax.iota(jnp.int32, 16)


def _read_count(cnt_v, w):
    rv = jnp.broadcast_to(w, (16,)).astype(jnp.int32)
    return plsc.load_gather(cnt_v, [rv, _iota16()])[0]


def _tc_matmul(x, W, Asd, apply_elu):
    n, din = x.shape
    dout = W.shape[1]
    bm = 1024
    nh = max(dout // 256, 1)
    dh = dout // nh

    def body(x_ref, w_ref, asd_ref, *out_refs):
        i = pl.program_id(0)
        xb = x_ref[...]
        if apply_elu:
            xb = jnp.where(xb > 0, xb, jnp.exp(jnp.minimum(xb, 0.0)) - 1.0)
        h = jnp.dot(xb, w_ref[...], preferred_element_type=jnp.float32)
        for t in range(nh):
            out_refs[t][...] = h[:, t * dh:(t + 1) * dh]
        sad = jnp.dot(h, asd_ref[...], preferred_element_type=jnp.float32)
        out_refs[nh][...] = sad

        @pl.when(i == 0)
        def _():
            out_refs[nh + 1][...] = jnp.full((8, 128), -1e30, jnp.float32)

        m = jnp.max(sad, axis=0)
        out_refs[nh + 1][...] = jnp.maximum(out_refs[nh + 1][...], m[None, :])

    grid = (n // bm,)
    out_shapes = ([jax.ShapeDtypeStruct((n, dh), jnp.float32) for _ in range(nh)]
                  + [jax.ShapeDtypeStruct((n, 128), jnp.float32),
                     jax.ShapeDtypeStruct((8, 128), jnp.float32)])
    out_specs = ([pl.BlockSpec((bm, dh), lambda i: (i, 0)) for _ in range(nh)]
                 + [pl.BlockSpec((bm, 128), lambda i: (i, 0)),
                    pl.BlockSpec((8, 128), lambda i: (0, 0))])
    outs = pl.pallas_call(
        body,
        grid=grid,
        in_specs=[pl.BlockSpec((bm, din), lambda i: (i, 0)),
                  pl.BlockSpec((din, dout), lambda i: (0, 0)),
                  pl.BlockSpec((dout, 128), lambda i: (0, 0))],
        out_specs=out_specs,
        out_shape=out_shapes,
    )(x, W, Asd)
    hs = outs[:nh]
    sad_full = outs[nh]
    astar16 = outs[nh + 1][0, :16]
    return hs, sad_full, astar16


def _sc_debug(ec, epad, stage):
    @functools.partial(
        pl.kernel,
        out_type=jax.ShapeDtypeStruct((NW, 16), jnp.int32),
        mesh=_mesh(),
        compiler_params=_SC_PARAMS,
        scratch_types=[pltpu.VMEM((CH,), jnp.int32),
                       pltpu.VMEM((2 * CH + 16,), jnp.int32),
                       pltpu.VMEM((16,), jnp.int32)],
    )
    def kern(ec_hbm, cnt_hbm, chunk_v, cbuf_v, misc_v):
        w = _wid()
        lo = w * NPW
        hi = lo + NPW
        acc = jnp.int32(0)
        if stage >= 10:
            pltpu.sync_copy(ec_hbm.at[pl.ds(0, CH)], chunk_v)
            v = chunk_v[pl.ds(0, 16)]
            acc = acc + v[0]
        if stage >= 11:
            v = chunk_v[pl.ds(pl.multiple_of(acc % 16 * 0 + 16, 16), 16)]
            acc = acc + v[0]
        if stage >= 12:
            ev = chunk_v[pl.ds(0, 16)]
            msk = (ev & 16383) >= lo
            npc = plsc.all_reduce_population_count(msk)
            if npc.ndim > 0:
                npc = npc[0]
            acc = acc + npc
        if stage >= 13:
            it = _iota16()
            ev = chunk_v[pl.ds(16, 16)]
            msk = ((ev & 16383) >= lo) & ((ev & 16383) < hi)
            key = jnp.where(msk, it, it + 16)
            sk, sv = plsc.sort_key_val(key, ev)
            ptr = acc % 16
            cbuf_v[pl.ds(ptr, 16)] = sv
            acc = acc + sk[0]
        misc_v[...] = jnp.broadcast_to(acc, (16,)).astype(jnp.int32)
        pltpu.sync_copy(misc_v, cnt_hbm.at[w])

    return kern(ec)


def _sc_compact(ec, epad):
    nouter = epad // CH

    @functools.partial(
        pl.kernel,
        out_type=(jax.ShapeDtypeStruct((NW, epad + CH), jnp.int32),
                  jax.ShapeDtypeStruct((NW, 16), jnp.int32)),
        mesh=_mesh(),
        compiler_params=_SC_PARAMS,
        scratch_types=[pltpu.VMEM((CH,), jnp.int32),
                       pltpu.VMEM((2 * CH + 16,), jnp.int32),
                       pltpu.VMEM((16,), jnp.int32)],
    )
    def kern(ec_hbm, ecc_hbm, cnt_hbm, chunk_v, cbuf_v, misc_v):
        w = _wid()
        lo = w * NPW
        hi = lo + NPW
        it = _iota16()

        def outer(o, carry):
            ptr, nfl = carry
            pltpu.sync_copy(ec_hbm.at[pl.ds(o * CH, CH)], chunk_v)

            def group(g, ptr):
                ev = chunk_v[pl.ds(g * 16, 16)]
                dv = ev & 16383
                msk = (dv >= lo) & (dv < hi)
                key = jnp.where(msk, it, it + 16)
                _, sv = plsc.sort_key_val(key, ev)
                cbuf_v[pl.ds(ptr, 16)] = sv
                npc = plsc.all_reduce_population_count(msk)
                if npc.ndim > 0:
                    npc = npc[0]
                return ptr + npc

            ptr = lax.fori_loop(0, CH // 16, group, ptr)
            pltpu.sync_copy(cbuf_v.at[pl.ds(0, CH)],
                            ecc_hbm.at[w, pl.ds(nfl * CH, CH)])
            full = ptr >= CH

            def shift(j, _):
                off = pl.multiple_of(j * 16, 16)
                t = cbuf_v[pl.ds(CH + off, 16)]
                h = cbuf_v[pl.ds(off, 16)]
                cbuf_v[pl.ds(off, 16)] = jnp.where(full, t, h)
                return 0

            lax.fori_loop(0, CH // 16, shift, 0)
            ptr = jnp.where(full, ptr - CH, ptr)
            nfl = jnp.where(full, nfl + 1, nfl)
            return (ptr, nfl)

        ptr, nfl = lax.fori_loop(0, nouter, outer,
                                 (jnp.int32(0), jnp.int32(0)))
        pltpu.sync_copy(cbuf_v.at[pl.ds(0, CH)],
                        ecc_hbm.at[w, pl.ds(nfl * CH, CH)])
        cnt = nfl * CH + ptr
        misc_v[...] = jnp.broadcast_to(cnt, (16,)).astype(jnp.int32)
        pltpu.sync_copy(misc_v, cnt_hbm.at[w])

    return kern(ec)


def _lrelu(x):
    return jnp.maximum(x, 0.2 * x)


def _sc_phase_b(ecc, counts, sad, astar, H, ecap):
    @functools.partial(
        pl.kernel,
        out_type=(jax.ShapeDtypeStruct((NW, ecap * H), jnp.float32),
                  jax.ShapeDtypeStruct((NW, NPW * H), jnp.float32)),
        mesh=_mesh(),
        compiler_params=_SC_PARAMS,
        scratch_types=[pltpu.VMEM((NW, 16), jnp.int32),
                       pltpu.VMEM((NPW, 128), jnp.float32),
                       pltpu.VMEM((NPW * H,), jnp.float32),
                       pltpu.VMEM((16,), jnp.float32),
                       pltpu.VMEM((K,), jnp.int32),
                       pltpu.VMEM((K,), jnp.int32),
                       pltpu.VMEM((K,), jnp.int32),
                       pltpu.VMEM((K, 128), jnp.float32),
                       pltpu.VMEM((K * H,), jnp.float32),
                       pltpu.SemaphoreType.DMA],
    )
    def kern(ecc_hbm, cnt_hbm, sad_hbm, astar_hbm, ex_hbm, den_hbm,
             cnt_v, sadl_v, den_v, astar_v, ec_v, si_v, dl_v, srow_v, ex_v,
             sem):
        w = _wid()
        lo = w * NPW
        it = _iota16()
        pltpu.sync_copy(cnt_hbm, cnt_v)
        pltpu.sync_copy(sad_hbm.at[pl.ds(lo, NPW)], sadl_v)
        pltpu.sync_copy(astar_hbm, astar_v)
        zf = jnp.zeros((16,), jnp.float32)

        def zinit(r, _):
            den_v[pl.ds(pl.multiple_of(r * 16, 16), 16)] = zf
            return 0

        lax.fori_loop(0, NPW * H // 16, zinit, 0)
        astar = astar_v[pl.ds(0, 16)]
        cnt = _read_count(cnt_v, w)
        nch = (cnt + (K - 1)) // K

        def chunk(j, _):
            off = pl.multiple_of(j * K, K)
            pltpu.sync_copy(ecc_hbm.at[w, pl.ds(off, K)], ec_v)
            for g in range(K // 16):
                ev = ec_v[pl.ds(g * 16, 16)]
                sv = jnp.clip(ev >> 14, 0, NPAD - 1)
                dv = jnp.clip((ev & 16383) - lo, 0, NPW - 1)
                si_v[pl.ds(g * 16, 16)] = sv
                dl_v[pl.ds(g * 16, 16)] = dv
            pltpu.async_copy(sad_hbm.at[si_v], srow_v, sem).wait()
            for g in range(K // 16):
                rowv = jnp.full((16,), g * 16, jnp.int32) + it
                dv = dl_v[pl.ds(g * 16, 16)]
                slot = off + g * 16 + it
                msk = slot < cnt
                for hd in range(H):
                    hc = jnp.full((16,), hd, jnp.int32)
                    s = plsc.load_gather(srow_v, [rowv, hc])
                    a = plsc.load_gather(sadl_v, [dv, hc + 8])
                    ash = astar[hd]
                    ex = jnp.exp(_lrelu(s + a) - _lrelu(ash + a))
                    ex = jnp.where(msk, ex, 0.0)
                    plsc.addupdate_scatter(den_v, [dv * H + hd], ex)
                    plsc.store_scatter(ex_v, [(rowv * H) + hd], ex)
            pltpu.sync_copy(ex_v, ex_hbm.at[w, pl.ds(off * H, K * H)])
            return 0

        lax.fori_loop(0, nch, chunk, 0)
        pltpu.sync_copy(den_v, den_hbm.at[w])

    return kern(ecc, counts, sad, astar)


def _sc_phase_c(ecc, counts, ex, den, hs, bias, H, dout, ecap):
    nh = len(hs)
    dh = dout // nh
    hh = dh // (dout // H)
    hw = dout // H

    @functools.partial(
        pl.kernel,
        out_type=jax.ShapeDtypeStruct((NPAD, dout), jnp.float32),
        mesh=_mesh(),
        compiler_params=_SC_PARAMS,
        scratch_types=[pltpu.VMEM((NW, 16), jnp.int32),
                       pltpu.VMEM((NPW * H,), jnp.float32),
                       pltpu.VMEM((K,), jnp.int32),
                       pltpu.VMEM((K * H,), jnp.float32),
                       pltpu.VMEM((2, K), jnp.int32),
                       pltpu.VMEM((2, K), jnp.int32),
                       pltpu.VMEM((2 * max(hh, 1), K), jnp.float32),
                       pltpu.VMEM((2, K, dh), jnp.float32),
                       pltpu.VMEM((NPW, dh), jnp.float32),
                       pltpu.VMEM((dh,), jnp.float32),
                       pltpu.SemaphoreType.DMA,
                       pltpu.SemaphoreType.DMA],
    )
    def kern(ecc_hbm, cnt_hbm, ex_hbm, den_hbm, *rest):
        h_hbms = rest[:nh]
        bias_hbm = rest[nh]
        out_hbm = rest[nh + 1]
        (cnt_v, rden_v, ec_v, ex_v, si_v, dl_v, attn_v, rows_v, acc_v,
         bias_v, sem0, sem1) = rest[nh + 2:]
        sems = (sem0, sem1)
        w = _wid()
        lo = w * NPW
        it = _iota16()
        pltpu.sync_copy(cnt_hbm, cnt_v)
        pltpu.sync_copy(den_hbm.at[w], rden_v)

        def rinit(r, _):
            off = pl.multiple_of(r * 16, 16)
            rden_v[pl.ds(off, 16)] = 1.0 / rden_v[pl.ds(off, 16)]
            return 0

        lax.fori_loop(0, NPW * H // 16, rinit, 0)
        cnt = _read_count(cnt_v, w)
        nch = (cnt + (K - 1)) // K
        nchp = (nch + 1) // 2

        for half in range(nh):
            pltpu.sync_copy(bias_hbm.at[pl.ds(half * dh, dh)], bias_v)

            def binit(r, _):
                rv = jnp.broadcast_to(r, (16,)).astype(jnp.int32)
                for v in range(dh // 16):
                    cv = jnp.full((16,), v * 16, jnp.int32) + it
                    plsc.store_scatter(acc_v, [rv, cv],
                                       bias_v[pl.ds(v * 16, 16)])
                return 0

            lax.fori_loop(0, NPW, binit, 0)

            def fetch(j, bi):
                off = pl.multiple_of(j * K, K)
                pltpu.sync_copy(ecc_hbm.at[w, pl.ds(off, K)], ec_v)
                pltpu.sync_copy(ex_hbm.at[w, pl.ds(off * H, K * H)], ex_v)

                def decode(g, _):
                    gb = pl.multiple_of(g * 16, 16)
                    ev = ec_v[pl.ds(gb, 16)]
                    sv = jnp.clip(ev >> 14, 0, NPAD - 1)
                    dv = jnp.clip((ev & 16383) - lo, 0, NPW - 1)
                    si_v[bi, pl.ds(gb, 16)] = sv
                    dl_v[bi, pl.ds(gb, 16)] = dv
                    rowv = jnp.broadcast_to(gb, (16,)).astype(jnp.int32) + it
                    msk = (off + gb + it) < cnt
                    for hd in range(hh):
                        ghd = half * hh + hd
                        exg = plsc.load_gather(ex_v, [rowv * H + ghd])
                        rdg = plsc.load_gather(rden_v, [dv * H + ghd])
                        attn_v[bi * hh + hd, pl.ds(gb, 16)] = jnp.where(
                            msk, exg * rdg, 0.0)
                    return 0

                lax.fori_loop(0, K // 16, decode, 0)
                pltpu.async_copy(h_hbms[half].at[si_v.at[bi]],
                                 rows_v.at[bi], sems[bi])

            def accum(bi):
                def accg(g, _):
                    gb = pl.multiple_of(g * 16, 16)
                    dlg = dl_v[bi, pl.ds(gb, 16)]
                    avs = [attn_v[bi * hh + hd, pl.ds(gb, 16)]
                           for hd in range(hh)]
                    for lane in range(16):
                        ev16 = jnp.broadcast_to(gb + lane, (16,)).astype(
                            jnp.int32)
                        rv = jnp.broadcast_to(dlg[lane], (16,)).astype(
                            jnp.int32)
                        for hd in range(hh):
                            a = avs[hd][lane]
                            for v in range(hw // 16):
                                col = hd * hw + v * 16
                                cv = jnp.full((16,), col, jnp.int32) + it
                                rvec = plsc.load_gather(
                                    rows_v, [jnp.full((16,), bi, jnp.int32),
                                             ev16, cv])
                                plsc.addupdate_scatter(
                                    acc_v, [rv, cv], a * rvec)
                    return 0

                lax.fori_loop(0, K // 16, accg, 0)

            def waitg(bi):
                pltpu.make_async_copy(h_hbms[half].at[si_v.at[bi]],
                                     rows_v.at[bi], sems[bi]).wait()

            fetch(jnp.int32(0), 0)

            def pair(p, _):
                for b in range(2):
                    j = p * 2 + b
                    fetch(j + 1, b ^ 1)
                    waitg(b)
                    accum(b)
                return 0

            lax.fori_loop(0, nchp, pair, 0)
            waitg(0)
            pltpu.sync_copy(
                acc_v, out_hbm.at[pl.ds(lo, NPW), pl.ds(half * dh, dh)])

    return kern(ecc, counts, ex, den, *hs, bias)


def _build_asd(a_s, a_d, dout):
    H, C = a_s.shape
    eye = jnp.eye(H, dtype=jnp.float32)
    As = (eye[:, None, :] * a_s[:, :, None]).reshape(H * C, H)
    Ad = (eye[:, None, :] * a_d[:, :, None]).reshape(H * C, H)
    Asd = jnp.zeros((dout, 128), jnp.float32)
    Asd = Asd.at[:, 0:H].set(As).at[:, 8:8 + H].set(Ad)
    return Asd


def _gat_layer(x, ecc, counts, W, a_s, a_d, b, H, apply_elu, ecap):
    dout = W.shape[1]
    hs, sad16, astar16 = _tc_matmul(x, W, _build_asd(a_s, a_d, dout),
                                    apply_elu)
    ex, den = _sc_phase_b(ecc, counts, sad16, astar16, H, ecap)
    out = _sc_phase_c(ecc, counts, ex, den, hs, b, H, dout, ecap)
    return out


def kernel(x, edge_index, W1, a1s, a1d, b1, W2, a2s, a2d, b2, W3, a3s, a3d,
           b3):
    n = x.shape[0]
    e = edge_index.shape[1]
    ep = e + n
    epad = ((ep + CH - 1) // CH) * CH
    ecap = epad + CH

    loop = jnp.arange(n, dtype=jnp.int32)
    src = jnp.concatenate([edge_index[0].astype(jnp.int32), loop,
                           jnp.zeros((epad - ep,), jnp.int32)])
    dst = jnp.concatenate([edge_index[1].astype(jnp.int32), loop,
                           jnp.full((epad - ep,), NPAD - 1, jnp.int32)])
    ec = (src << 14) | dst

    if 10 <= _DEBUG_STAGE < 20:
        cnts = _sc_debug(ec, epad, _DEBUG_STAGE)
        return jnp.zeros((n, 128), jnp.float32) + cnts[0, 0].astype(
            jnp.float32)
    ecc, counts = _sc_compact(ec, epad)
    if _DEBUG_STAGE == 1:
        return jnp.zeros((n, 128), jnp.float32) + counts[0, 0].astype(
            jnp.float32) + ecc[0, 0].astype(jnp.float32)

    xpad = jnp.concatenate(
        [x, jnp.zeros((NPAD - n, x.shape[1]), jnp.float32)])

    if _DEBUG_STAGE == 2:
        hs, sad16, astar16 = _tc_matmul(xpad, W1, _build_asd(a1s, a1d, 512),
                                        False)
        ex, den = _sc_phase_b(ecc, counts, sad16, astar16, 8, ecap)
        return jnp.zeros((n, 128), jnp.float32) + ex[0, 0] + den[0, 0]
    if _DEBUG_STAGE == 3:
        h1 = _gat_layer(xpad, ecc, counts, W1, a1s, a1d, b1, 8, False, ecap)
        return h1[:n, :128]

    h1 = _gat_layer(xpad, ecc, counts, W1, a1s, a1d, b1, 8, False, ecap)
    h2 = _gat_layer(h1, ecc, counts, W2, a2s, a2d, b2, 8, True, ecap)
    h3 = _gat_layer(h2, ecc, counts, W3, a3s, a3d, b3, 1, True, ecap)
    return h3[:n]


# speedup vs baseline: 1.6326x; 1.6326x over previous
import functools

import jax
import jax.numpy as jnp
from jax import lax
from jax.experimental import pallas as pl
from jax.experimental.pallas import tpu as pltpu
from jax.experimental.pallas import tpu_sc as plsc

N = 10000
NW = 32
NPW = 320
NPAD = NW * NPW
CH = 1024
K = 64
NC = 2
NS = 16
_DEBUG_STAGE = 0

_mesh = lambda: plsc.VectorSubcoreMesh(
    core_axis_name="c", subcore_axis_name="s", num_cores=NC, num_subcores=NS)
_SC_PARAMS = pltpu.CompilerParams(needs_layout_passes=False)


def _wid():
    return lax.axis_index("s") * NC + lax.axis_index("c")


def _iota16():
    return lax.iota(jnp.int32, 16)


def _read_count(cnt_v, w):
    rv = jnp.broadcast_to(w, (16,)).astype(jnp.int32)
    return plsc.load_gather(cnt_v, [rv, _iota16()])[0]


def _tc_matmul(x, W, Asd, apply_elu):
    n, din = x.shape
    dout = W.shape[1]
    bm = 1024
    nh = max(dout // 256, 1)
    dh = dout // nh

    def body(x_ref, w_ref, asd_ref, *out_refs):
        i = pl.program_id(0)
        xb = x_ref[...]
        if apply_elu:
            xb = jnp.where(xb > 0, xb, jnp.exp(jnp.minimum(xb, 0.0)) - 1.0)
        h = jnp.dot(xb, w_ref[...], preferred_element_type=jnp.float32)
        for t in range(nh):
            out_refs[t][...] = h[:, t * dh:(t + 1) * dh]
        sad = jnp.dot(h, asd_ref[...], preferred_element_type=jnp.float32)
        out_refs[nh][...] = sad

        @pl.when(i == 0)
        def _():
            out_refs[nh + 1][...] = jnp.full((8, 128), -1e30, jnp.float32)

        m = jnp.max(sad, axis=0)
        out_refs[nh + 1][...] = jnp.maximum(out_refs[nh + 1][...], m[None, :])

    grid = (n // bm,)
    out_shapes = ([jax.ShapeDtypeStruct((n, dh), jnp.float32) for _ in range(nh)]
                  + [jax.ShapeDtypeStruct((n, 128), jnp.float32),
                     jax.ShapeDtypeStruct((8, 128), jnp.float32)])
    out_specs = ([pl.BlockSpec((bm, dh), lambda i: (i, 0)) for _ in range(nh)]
                 + [pl.BlockSpec((bm, 128), lambda i: (i, 0)),
                    pl.BlockSpec((8, 128), lambda i: (0, 0))])
    outs = pl.pallas_call(
        body,
        grid=grid,
        in_specs=[pl.BlockSpec((bm, din), lambda i: (i, 0)),
                  pl.BlockSpec((din, dout), lambda i: (0, 0)),
                  pl.BlockSpec((dout, 128), lambda i: (0, 0))],
        out_specs=out_specs,
        out_shape=out_shapes,
    )(x, W, Asd)
    hs = outs[:nh]
    sad_full = outs[nh]
    astar16 = outs[nh + 1][0, :16]
    return hs, sad_full, astar16


def _sc_debug(ec, epad, stage):
    @functools.partial(
        pl.kernel,
        out_type=jax.ShapeDtypeStruct((NW, 16), jnp.int32),
        mesh=_mesh(),
        compiler_params=_SC_PARAMS,
        scratch_types=[pltpu.VMEM((CH,), jnp.int32),
                       pltpu.VMEM((2 * CH + 16,), jnp.int32),
                       pltpu.VMEM((16,), jnp.int32)],
    )
    def kern(ec_hbm, cnt_hbm, chunk_v, cbuf_v, misc_v):
        w = _wid()
        lo = w * NPW
        hi = lo + NPW
        acc = jnp.int32(0)
        if stage >= 10:
            pltpu.sync_copy(ec_hbm.at[pl.ds(0, CH)], chunk_v)
            v = chunk_v[pl.ds(0, 16)]
            acc = acc + v[0]
        if stage >= 11:
            v = chunk_v[pl.ds(pl.multiple_of(acc % 16 * 0 + 16, 16), 16)]
            acc = acc + v[0]
        if stage >= 12:
            ev = chunk_v[pl.ds(0, 16)]
            msk = (ev & 16383) >= lo
            npc = plsc.all_reduce_population_count(msk)
            if npc.ndim > 0:
                npc = npc[0]
            acc = acc + npc
        if stage >= 13:
            it = _iota16()
            ev = chunk_v[pl.ds(16, 16)]
            msk = ((ev & 16383) >= lo) & ((ev & 16383) < hi)
            key = jnp.where(msk, it, it + 16)
            sk, sv = plsc.sort_key_val(key, ev)
            ptr = acc % 16
            cbuf_v[pl.ds(ptr, 16)] = sv
            acc = acc + sk[0]
        misc_v[...] = jnp.broadcast_to(acc, (16,)).astype(jnp.int32)
        pltpu.sync_copy(misc_v, cnt_hbm.at[w])

    return kern(ec)


def _sc_compact(ec, epad):
    nouter = epad // CH

    @functools.partial(
        pl.kernel,
        out_type=(jax.ShapeDtypeStruct((NW, epad + CH), jnp.int32),
                  jax.ShapeDtypeStruct((NW, 16), jnp.int32)),
        mesh=_mesh(),
        compiler_params=_SC_PARAMS,
        scratch_types=[pltpu.VMEM((CH,), jnp.int32),
                       pltpu.VMEM((2 * CH + 16,), jnp.int32),
                       pltpu.VMEM((16,), jnp.int32)],
    )
    def kern(ec_hbm, ecc_hbm, cnt_hbm, chunk_v, cbuf_v, misc_v):
        w = _wid()
        lo = w * NPW
        hi = lo + NPW
        it = _iota16()

        def outer(o, carry):
            ptr, nfl = carry
            pltpu.sync_copy(ec_hbm.at[pl.ds(o * CH, CH)], chunk_v)

            def group(g, ptr):
                ev = chunk_v[pl.ds(g * 16, 16)]
                dv = ev & 16383
                msk = (dv >= lo) & (dv < hi)
                key = jnp.where(msk, it, it + 16)
                _, sv = plsc.sort_key_val(key, ev)
                cbuf_v[pl.ds(ptr, 16)] = sv
                npc = plsc.all_reduce_population_count(msk)
                if npc.ndim > 0:
                    npc = npc[0]
                return ptr + npc

            ptr = lax.fori_loop(0, CH // 16, group, ptr)
            pltpu.sync_copy(cbuf_v.at[pl.ds(0, CH)],
                            ecc_hbm.at[w, pl.ds(nfl * CH, CH)])
            full = ptr >= CH

            def shift(j, _):
                off = pl.multiple_of(j * 16, 16)
                t = cbuf_v[pl.ds(CH + off, 16)]
                h = cbuf_v[pl.ds(off, 16)]
                cbuf_v[pl.ds(off, 16)] = jnp.where(full, t, h)
                return 0

            lax.fori_loop(0, CH // 16, shift, 0)
            ptr = jnp.where(full, ptr - CH, ptr)
            nfl = jnp.where(full, nfl + 1, nfl)
            return (ptr, nfl)

        ptr, nfl = lax.fori_loop(0, nouter, outer,
                                 (jnp.int32(0), jnp.int32(0)))
        pltpu.sync_copy(cbuf_v.at[pl.ds(0, CH)],
                        ecc_hbm.at[w, pl.ds(nfl * CH, CH)])
        cnt = nfl * CH + ptr
        misc_v[...] = jnp.broadcast_to(cnt, (16,)).astype(jnp.int32)
        pltpu.sync_copy(misc_v, cnt_hbm.at[w])

    return kern(ec)


def _lrelu(x):
    return jnp.maximum(x, 0.2 * x)


def _sc_phase_b(ecc, counts, sad, astar, H, ecap):
    ncap = ecap // K

    @functools.partial(
        pl.kernel,
        out_type=(jax.ShapeDtypeStruct((NW, ncap, H, K), jnp.float32),
                  jax.ShapeDtypeStruct((NW, NPW * H), jnp.float32)),
        mesh=_mesh(),
        compiler_params=_SC_PARAMS,
        scratch_types=[pltpu.VMEM((NW, 16), jnp.int32),
                       pltpu.VMEM((NPW, 128), jnp.float32),
                       pltpu.VMEM((NPW * H,), jnp.float32),
                       pltpu.VMEM((16,), jnp.float32),
                       pltpu.VMEM((K,), jnp.int32),
                       pltpu.VMEM((K,), jnp.int32),
                       pltpu.VMEM((K,), jnp.int32),
                       pltpu.VMEM((K, 128), jnp.float32),
                       pltpu.VMEM((H, K), jnp.float32),
                       pltpu.SemaphoreType.DMA],
    )
    def kern(ecc_hbm, cnt_hbm, sad_hbm, astar_hbm, ex_hbm, den_hbm,
             cnt_v, sadl_v, den_v, astar_v, ec_v, si_v, dl_v, srow_v, ex_v,
             sem):
        w = _wid()
        lo = w * NPW
        it = _iota16()
        pltpu.sync_copy(cnt_hbm, cnt_v)
        pltpu.sync_copy(sad_hbm.at[pl.ds(lo, NPW)], sadl_v)
        pltpu.sync_copy(astar_hbm, astar_v)
        zf = jnp.zeros((16,), jnp.float32)

        def zinit(r, _):
            den_v[pl.ds(pl.multiple_of(r * 16, 16), 16)] = zf
            return 0

        lax.fori_loop(0, NPW * H // 16, zinit, 0)
        astar = astar_v[pl.ds(0, 16)]
        cnt = _read_count(cnt_v, w)
        nch = (cnt + (K - 1)) // K

        def chunk(j, _):
            off = pl.multiple_of(j * K, K)
            pltpu.sync_copy(ecc_hbm.at[w, pl.ds(off, K)], ec_v)
            for g in range(K // 16):
                ev = ec_v[pl.ds(g * 16, 16)]
                sv = jnp.clip(ev >> 14, 0, NPAD - 1)
                dv = jnp.clip((ev & 16383) - lo, 0, NPW - 1)
                si_v[pl.ds(g * 16, 16)] = sv
                dl_v[pl.ds(g * 16, 16)] = dv
            pltpu.async_copy(sad_hbm.at[si_v], srow_v, sem).wait()
            for g in range(K // 16):
                rowv = jnp.full((16,), g * 16, jnp.int32) + it
                dv = dl_v[pl.ds(g * 16, 16)]
                slot = off + g * 16 + it
                msk = slot < cnt
                for hd in range(H):
                    hc = jnp.full((16,), hd, jnp.int32)
                    s = plsc.load_gather(srow_v, [rowv, hc])
                    a = plsc.load_gather(sadl_v, [dv, hc + 8])
                    ash = astar[hd]
                    ex = jnp.exp(_lrelu(s + a) - _lrelu(ash + a))
                    ex = jnp.where(msk, ex, 0.0)
                    plsc.addupdate_scatter(den_v, [dv * H + hd], ex)
                    ex_v[hd, pl.ds(g * 16, 16)] = ex
            pltpu.sync_copy(ex_v, ex_hbm.at[w, j])
            return 0

        lax.fori_loop(0, nch, chunk, 0)
        pltpu.sync_copy(den_v, den_hbm.at[w])

    return kern(ecc, counts, sad, astar)


def _sc_phase_c(ecc, counts, ex, den, hs, bias, H, dout, ecap):
    nh = len(hs)
    dh = dout // nh
    hh = dh // (dout // H)
    hw = dout // H

    @functools.partial(
        pl.kernel,
        out_type=jax.ShapeDtypeStruct((NPAD, dout), jnp.float32),
        mesh=_mesh(),
        compiler_params=_SC_PARAMS,
        scratch_types=[pltpu.VMEM((NW, 16), jnp.int32),
                       pltpu.VMEM((NPW * H,), jnp.float32),
                       pltpu.VMEM((K,), jnp.int32),
                       pltpu.VMEM((H, K), jnp.float32),
                       pltpu.VMEM((K,), jnp.int32),
                       pltpu.VMEM((K,), jnp.int32),
                       pltpu.VMEM((K,), jnp.int32),
                       pltpu.VMEM((K,), jnp.int32),
                       pltpu.VMEM((max(hh, 1), K), jnp.float32),
                       pltpu.VMEM((max(hh, 1), K), jnp.float32),
                       pltpu.VMEM((K, dh), jnp.float32),
                       pltpu.VMEM((K, dh), jnp.float32),
                       pltpu.VMEM((NPW, dh), jnp.float32),
                       pltpu.VMEM((dh,), jnp.float32),
                       pltpu.SemaphoreType.DMA,
                       pltpu.SemaphoreType.DMA],
    )
    def kern(ecc_hbm, cnt_hbm, ex_hbm, den_hbm, *rest):
        h_hbms = rest[:nh]
        bias_hbm = rest[nh]
        out_hbm = rest[nh + 1]
        (cnt_v, rden_v, ec_v, ex_v, si0, si1, dl0, dl1, at0, at1,
         rw0, rw1, acc_v, bias_v, sem0, sem1) = rest[nh + 2:]
        sis = (si0, si1)
        dls = (dl0, dl1)
        ats = (at0, at1)
        rws = (rw0, rw1)
        sems = (sem0, sem1)
        w = _wid()
        lo = w * NPW
        it = _iota16()
        pltpu.sync_copy(cnt_hbm, cnt_v)
        pltpu.sync_copy(den_hbm.at[w], rden_v)

        def rinit(r, _):
            off = pl.multiple_of(r * 16, 16)
            rden_v[pl.ds(off, 16)] = 1.0 / rden_v[pl.ds(off, 16)]
            return 0

        lax.fori_loop(0, NPW * H // 16, rinit, 0)
        cnt = _read_count(cnt_v, w)
        nch = (cnt + (K - 1)) // K
        nchp = (nch + 1) // 2

        for half in range(nh):
            pltpu.sync_copy(bias_hbm.at[pl.ds(half * dh, dh)], bias_v)

            def binit(r, _):
                rv = jnp.broadcast_to(r, (16,)).astype(jnp.int32)
                for v in range(dh // 16):
                    cv = jnp.full((16,), v * 16, jnp.int32) + it
                    plsc.store_scatter(acc_v, [rv, cv],
                                       bias_v[pl.ds(v * 16, 16)])
                return 0

            lax.fori_loop(0, NPW, binit, 0)

            def fetch(j, bi):
                off = pl.multiple_of(j * K, K)
                pltpu.sync_copy(ecc_hbm.at[w, pl.ds(off, K)], ec_v)
                pltpu.sync_copy(ex_hbm.at[w, j], ex_v)

                def decode(g, _):
                    gb = pl.multiple_of(g * 16, 16)
                    ev = ec_v[pl.ds(gb, 16)]
                    sv = jnp.clip(ev >> 14, 0, NPAD - 1)
                    dv = jnp.clip((ev & 16383) - lo, 0, NPW - 1)
                    sis[bi][pl.ds(gb, 16)] = sv
                    dls[bi][pl.ds(gb, 16)] = dv
                    msk = (off + gb + it) < cnt
                    for hd in range(hh):
                        ghd = half * hh + hd
                        exg = ex_v[ghd, pl.ds(gb, 16)]
                        rdg = plsc.load_gather(rden_v, [dv * H + ghd])
                        ats[bi][hd, pl.ds(gb, 16)] = jnp.where(
                            msk, exg * rdg, 0.0)
                    return 0

                lax.fori_loop(0, K // 16, decode, 0)
                pltpu.async_copy(h_hbms[half].at[sis[bi]], rws[bi], sems[bi])

            def accum(bi):
                def accg(g, _):
                    gb = pl.multiple_of(g * 16, 16)
                    dlg = dls[bi][pl.ds(gb, 16)]
                    avs = [ats[bi][hd, pl.ds(gb, 16)] for hd in range(hh)]
                    for lane in range(16):
                        e = gb + lane
                        dl = dlg[lane]
                        for hd in range(hh):
                            a = avs[hd][lane]
                            for v in range(hw // 16):
                                col = hd * hw + v * 16
                                x = rws[bi][e, pl.ds(col, 16)]
                                plsc.addupdate(
                                    acc_v.at[dl, pl.ds(col, 16)], a * x)
                    return 0

                lax.fori_loop(0, K // 16, accg, 0)

            def waitg(bi):
                pltpu.make_async_copy(h_hbms[half].at[sis[bi]],
                                      rws[bi], sems[bi]).wait()

            fetch(jnp.int32(0), 0)

            def pair(p, _):
                for b in range(2):
                    j = p * 2 + b
                    fetch(j + 1, b ^ 1)
                    waitg(b)
                    accum(b)
                return 0

            lax.fori_loop(0, nchp, pair, 0)
            waitg(0)
            pltpu.sync_copy(
                acc_v, out_hbm.at[pl.ds(lo, NPW), pl.ds(half * dh, dh)])

    return kern(ecc, counts, ex, den, *hs, bias)


def _build_asd(a_s, a_d, dout):
    H, C = a_s.shape
    eye = jnp.eye(H, dtype=jnp.float32)
    As = (eye[:, None, :] * a_s[:, :, None]).reshape(H * C, H)
    Ad = (eye[:, None, :] * a_d[:, :, None]).reshape(H * C, H)
    Asd = jnp.zeros((dout, 128), jnp.float32)
    Asd = Asd.at[:, 0:H].set(As).at[:, 8:8 + H].set(Ad)
    return Asd


def _gat_layer(x, ecc, counts, W, a_s, a_d, b, H, apply_elu, ecap):
    dout = W.shape[1]
    hs, sad16, astar16 = _tc_matmul(x, W, _build_asd(a_s, a_d, dout),
                                    apply_elu)
    ex, den = _sc_phase_b(ecc, counts, sad16, astar16, H, ecap)
    out = _sc_phase_c(ecc, counts, ex, den, hs, b, H, dout, ecap)
    return out


def kernel(x, edge_index, W1, a1s, a1d, b1, W2, a2s, a2d, b2, W3, a3s, a3d,
           b3):
    n = x.shape[0]
    e = edge_index.shape[1]
    ep = e + n
    epad = ((ep + CH - 1) // CH) * CH
    ecap = epad + CH

    loop = jnp.arange(n, dtype=jnp.int32)
    src = jnp.concatenate([edge_index[0].astype(jnp.int32), loop,
                           jnp.zeros((epad - ep,), jnp.int32)])
    dst = jnp.concatenate([edge_index[1].astype(jnp.int32), loop,
                           jnp.full((epad - ep,), NPAD - 1, jnp.int32)])
    ec = (src << 14) | dst

    if 10 <= _DEBUG_STAGE < 20:
        cnts = _sc_debug(ec, epad, _DEBUG_STAGE)
        return jnp.zeros((n, 128), jnp.float32) + cnts[0, 0].astype(
            jnp.float32)
    ecc, counts = _sc_compact(ec, epad)
    if _DEBUG_STAGE == 1:
        return jnp.zeros((n, 128), jnp.float32) + counts[0, 0].astype(
            jnp.float32) + ecc[0, 0].astype(jnp.float32)

    xpad = jnp.concatenate(
        [x, jnp.zeros((NPAD - n, x.shape[1]), jnp.float32)])

    if _DEBUG_STAGE == 2:
        hs, sad16, astar16 = _tc_matmul(xpad, W1, _build_asd(a1s, a1d, 512),
                                        False)
        ex, den = _sc_phase_b(ecc, counts, sad16, astar16, 8, ecap)
        return jnp.zeros((n, 128), jnp.float32) + ex[0, 0] + den[0, 0]
    if _DEBUG_STAGE == 3:
        h1 = _gat_layer(xpad, ecc, counts, W1, a1s, a1d, b1, 8, False, ecap)
        return h1[:n, :128]

    h1 = _gat_layer(xpad, ecc, counts, W1, a1s, a1d, b1, 8, False, ecap)
    h2 = _gat_layer(h1, ecc, counts, W2, a2s, a2d, b2, 8, True, ecap)
    h3 = _gat_layer(h2, ecc, counts, W3, a3s, a3d, b3, 1, True, ecap)
    return h3[:n]
